# baseline (speedup 1.0000x reference)
import jax
import jax.numpy as jnp
from jax import lax
from jax.experimental import pallas as pl
from jax.experimental.pallas import tpu as pltpu
from jax.experimental.pallas import tpu_sc as plsc

_NC = 2
_NS = 16
_NW = _NC * _NS

_ROWS, _COLS = 4096, 200
_D = 64
_DP = 128
_RB = _ROWS // 128
_NBLK = _COLS * _RB
_BLK_PER_W = _NBLK // _NW


def _gather_body(xt_hbm, tab_hbm, out_hbm, idx_v, rows_v, tbuf_v, isem, gsem, wsem):
    wid = lax.axis_index("s") * _NC + lax.axis_index("c")
    base = wid * _BLK_PER_W
    iota = lax.iota(jnp.int32, 16)

    @pl.loop(0, _BLK_PER_W)
    def _blk(i):
        blk = base + i
        c = blk // _RB
        rb = blk % _RB
        pltpu.sync_copy(xt_hbm.at[c, pl.ds(rb * 128, 128)], idx_v)
        pltpu.async_copy(tab_hbm.at[idx_v], rows_v, gsem).wait()
        for fh in range(8):
            for fl in range(8):
                cvec = iota * 0 + (fh * 8 + fl)
                for r8 in range(8):
                    rvec = iota + r8 * 16
                    v = plsc.load_gather(rows_v, [rvec, cvec])
                    tbuf_v[fh, fl, pl.ds(r8 * 16, 16)] = v
        pltpu.async_copy(tbuf_v, out_hbm.at[c, :, rb], wsem).wait()


_mesh = plsc.VectorSubcoreMesh(core_axis_name="c", subcore_axis_name="s")

_gather = pl.kernel(
    _gather_body,
    out_type=jax.ShapeDtypeStruct((_COLS, 8, _RB, 8, 128), jnp.float32),
    mesh=_mesh,
    compiler_params=pltpu.CompilerParams(
        use_tc_tiling_on_sc=False, needs_layout_passes=False),
    scratch_types=[
        pltpu.VMEM((128,), jnp.int32),
        pltpu.VMEM((128, _DP), jnp.float32),
        pltpu.VMEM((8, 8, 128), jnp.float32),
        pltpu.SemaphoreType.DMA,
        pltpu.SemaphoreType.DMA,
        pltpu.SemaphoreType.DMA,
    ],
)


def kernel(x, table):
    xt = x.T.astype(jnp.int32)
    tab = jnp.pad(table, ((0, 0), (0, _DP - _D)))
    out5 = _gather(xt, tab)
    return out5.transpose(2, 4, 0, 1, 3).reshape(_ROWS, _COLS, _D)
